# baseline (speedup 1.0000x reference)
import functools

import jax
import jax.numpy as jnp
from jax import lax
from jax.experimental import pallas as pl
from jax.experimental.pallas import tpu as pltpu
from jax.experimental.pallas import tpu_sc as plsc

N = 100000
E = 1600000
D = 32
H = 16

NSUB = 16
CHUNK = 512
SS = 4

NP = 100352
TR = NP // NSUB
ZR = 128
EP = 1638400
NCT = EP // (SS * CHUNK)
NSUP = NCT // NSUB

BR = 2048
_MESH = plsc.VectorSubcoreMesh(
    core_axis_name="c", subcore_axis_name="s", num_cores=2, num_subcores=NSUB
)


def _agg_body(gidx_hbm, sidx_hbm, tlo_hbm, thi_hbm, out_hbm,
              acc_sh, gi_v, si_v, rows_v, zb_v,
              semi0, semi1, semg0, semg1):
    c = lax.axis_index("c")
    s = lax.axis_index("s")
    semi = (semi0, semi1)
    semg = (semg0, semg1)

    def _zero(j, carry):
        zb_v[j, :] = jnp.zeros((16,), jnp.float32)
        return carry

    lax.fori_loop(0, ZR, _zero, 0)
    r0 = s * TR
    for t in range(TR // ZR):
        pltpu.async_copy(zb_v, acc_sh.at[pl.ds(r0 + t * ZR, ZR)], semg0)
    for t in range(TR // ZR):
        pltpu.make_async_copy(zb_v, acc_sh.at[pl.ds(r0 + t * ZR, ZR)],
                              semg0).wait()
    plsc.subcore_barrier()

    def _load_super(p, j):
        row = s * NSUP + j
        pltpu.async_copy(gidx_hbm.at[row], gi_v.at[p], semi[p])
        pltpu.async_copy(sidx_hbm.at[row], si_v.at[p], semi[p])

    def _drain_super(p):
        pltpu.make_async_copy(gidx_hbm.at[0], gi_v.at[p], semi[p]).wait()
        pltpu.make_async_copy(sidx_hbm.at[0], si_v.at[p], semi[p]).wait()

    def _fire_g(p, m, b):
        @pl.when(c == 0)
        def _():
            pltpu.async_copy(tlo_hbm.at[gi_v.at[p, m]], rows_v.at[b], semg[b])

        @pl.when(c == 1)
        def _():
            pltpu.async_copy(thi_hbm.at[gi_v.at[p, m]], rows_v.at[b], semg[b])

    def _drain_g(p, m, b):
        pltpu.make_async_copy(tlo_hbm.at[gi_v.at[p, m]], rows_v.at[b],
                              semg[b]).wait()

    def _do_super(p, j):
        @pl.when(j + 1 < NSUP)
        def _():
            _load_super(1 - p, j + 1)

        _drain_super(p)
        _fire_g(p, 0, 0)
        _fire_g(p, 1, 1)
        for m in range(SS):
            b = m % 2
            _drain_g(p, m, b)
            pltpu.sync_copy(rows_v.at[b], acc_sh.at[si_v.at[p, m]], add=True)
            if m + 2 < SS:
                _fire_g(p, m + 2, b)

    _load_super(0, 0)

    def _iter(it, carry):
        _do_super(0, 2 * it)
        _do_super(1, 2 * it + 1)
        return carry

    lax.fori_loop(0, NSUP // 2, _iter, 0)
    plsc.subcore_barrier()

    @pl.when(c == 0)
    def _():
        pltpu.sync_copy(acc_sh.at[pl.ds(r0, TR)],
                        out_hbm.at[pl.ds(r0, TR), pl.ds(0, 16)])

    @pl.when(c == 1)
    def _():
        pltpu.sync_copy(acc_sh.at[pl.ds(r0, TR)],
                        out_hbm.at[pl.ds(r0, TR), pl.ds(16, 16)])


_agg_call = functools.partial(
    pl.kernel,
    out_type=jax.ShapeDtypeStruct((NP, D), jnp.float32),
    mesh=_MESH,
    scratch_types=[
        pltpu.VMEM_SHARED((NP, H), jnp.float32),
        pltpu.VMEM((2, SS, CHUNK), jnp.int32),
        pltpu.VMEM((2, SS, CHUNK), jnp.int32),
        pltpu.VMEM((2, CHUNK, H), jnp.float32),
        pltpu.VMEM((ZR, H), jnp.float32),
        pltpu.SemaphoreType.DMA,
        pltpu.SemaphoreType.DMA,
        pltpu.SemaphoreType.DMA,
        pltpu.SemaphoreType.DMA,
    ],
    compiler_params=pltpu.CompilerParams(use_tc_tiling_on_sc=False),
)(_agg_body)


def _cnt_body(su_hbm, sd_hbm, out_hbm, acc_sh, si_v, ones_v, zb_v,
              semi0, semi1):
    c = lax.axis_index("c")
    s = lax.axis_index("s")
    semi = (semi0, semi1)

    def _zero(j, carry):
        zb_v[j, :] = jnp.zeros((16,), jnp.float32)
        return carry

    lax.fori_loop(0, ZR, _zero, 0)

    def _one(j, carry):
        ones_v[j, :] = jnp.ones((16,), jnp.float32)
        return carry

    lax.fori_loop(0, CHUNK, _one, 0)
    r0 = s * TR
    for t in range(TR // ZR):
        pltpu.async_copy(zb_v, acc_sh.at[pl.ds(r0 + t * ZR, ZR)], semi0)
    for t in range(TR // ZR):
        pltpu.make_async_copy(zb_v, acc_sh.at[pl.ds(r0 + t * ZR, ZR)],
                              semi0).wait()
    plsc.subcore_barrier()

    def _load_super(p, j):
        row = s * NSUP + j

        @pl.when(c == 0)
        def _():
            pltpu.async_copy(su_hbm.at[row], si_v.at[p], semi[p])

        @pl.when(c == 1)
        def _():
            pltpu.async_copy(sd_hbm.at[row], si_v.at[p], semi[p])

    def _drain_super(p):
        pltpu.make_async_copy(su_hbm.at[0], si_v.at[p], semi[p]).wait()

    def _do_super(p, j):
        @pl.when(j + 1 < NSUP)
        def _():
            _load_super(1 - p, j + 1)

        _drain_super(p)
        for m in range(SS):
            pltpu.sync_copy(ones_v, acc_sh.at[si_v.at[p, m]], add=True)

    _load_super(0, 0)

    def _iter(it, carry):
        _do_super(0, 2 * it)
        _do_super(1, 2 * it + 1)
        return carry

    lax.fori_loop(0, NSUP // 2, _iter, 0)
    plsc.subcore_barrier()

    @pl.when(c == 0)
    def _():
        pltpu.sync_copy(acc_sh.at[pl.ds(r0, TR)],
                        out_hbm.at[pl.ds(r0, TR), pl.ds(0, 16)])

    @pl.when(c == 1)
    def _():
        pltpu.sync_copy(acc_sh.at[pl.ds(r0, TR)],
                        out_hbm.at[pl.ds(r0, TR), pl.ds(16, 16)])


_cnt_call = functools.partial(
    pl.kernel,
    out_type=jax.ShapeDtypeStruct((NP, D), jnp.float32),
    mesh=_MESH,
    scratch_types=[
        pltpu.VMEM_SHARED((NP, H), jnp.float32),
        pltpu.VMEM((2, SS, CHUNK), jnp.int32),
        pltpu.VMEM((CHUNK, H), jnp.float32),
        pltpu.VMEM((ZR, H), jnp.float32),
        pltpu.SemaphoreType.DMA,
        pltpu.SemaphoreType.DMA,
    ],
    compiler_params=pltpu.CompilerParams(use_tc_tiling_on_sc=False),
)(_cnt_body)


def _dense_body(agg_ref, cnt_ref, xlo_ref, xhi_ref, wl_ref, bl_ref, wr_ref,
                lo_ref, hi_ref, *, col):
    cnt = jnp.maximum(cnt_ref[:, col:col + 1], 1.0)
    agg = agg_ref[...] / cnt
    xd = jnp.concatenate([xlo_ref[...], xhi_ref[...]], axis=1)
    o = (jnp.dot(agg, wl_ref[...], preferred_element_type=jnp.float32)
         + bl_ref[...]
         + jnp.dot(xd, wr_ref[...], preferred_element_type=jnp.float32))
    nrm = jnp.sqrt(jnp.sum(o * o, axis=-1, keepdims=True))
    h = jnp.maximum(o / jnp.maximum(nrm, 1e-12), 0.0)
    lo_ref[...] = h[:, :H]
    hi_ref[...] = h[:, H:]


def _dense_final_body(agg_ref, cnt_ref, xlo_ref, xhi_ref, wl_ref, bl_ref,
                      wr_ref, wo_ref, bo_ref, out_ref, *, col):
    cnt = jnp.maximum(cnt_ref[:, col:col + 1], 1.0)
    agg = agg_ref[...] / cnt
    xd = jnp.concatenate([xlo_ref[...], xhi_ref[...]], axis=1)
    o = (jnp.dot(agg, wl_ref[...], preferred_element_type=jnp.float32)
         + bl_ref[...]
         + jnp.dot(xd, wr_ref[...], preferred_element_type=jnp.float32))
    nrm = jnp.sqrt(jnp.sum(o * o, axis=-1, keepdims=True))
    h = jnp.maximum(o / jnp.maximum(nrm, 1e-12), 0.0)
    e = jnp.dot(h, wo_ref[...], preferred_element_type=jnp.float32) + bo_ref[...]
    nrm2 = jnp.sqrt(jnp.sum(e * e, axis=-1, keepdims=True))
    out_ref[...] = e / jnp.maximum(nrm2, 1e-12)


_ROW_SPEC = pl.BlockSpec((BR, D), lambda i: (i, 0))
_HALF_SPEC = pl.BlockSpec((BR, H), lambda i: (i, 0))
_W_SPEC = pl.BlockSpec((D, D), lambda i: (0, 0))
_B_SPEC = pl.BlockSpec((1, D), lambda i: (0, 0))


def _dense(agg, cnt, xlo, xhi, wl, bl, wr, col):
    return pl.pallas_call(
        functools.partial(_dense_body, col=col),
        grid=(NP // BR,),
        in_specs=[_ROW_SPEC, _ROW_SPEC, _HALF_SPEC, _HALF_SPEC,
                  _W_SPEC, _B_SPEC, _W_SPEC],
        out_specs=[_HALF_SPEC, _HALF_SPEC],
        out_shape=[jax.ShapeDtypeStruct((NP, H), jnp.float32),
                   jax.ShapeDtypeStruct((NP, H), jnp.float32)],
    )(agg, cnt, xlo, xhi, wl, bl.reshape(1, D), wr)


def _dense_final(agg, cnt, xlo, xhi, wl, bl, wr, wo, bo, col):
    return pl.pallas_call(
        functools.partial(_dense_final_body, col=col),
        grid=(NP // BR,),
        in_specs=[_ROW_SPEC, _ROW_SPEC, _HALF_SPEC, _HALF_SPEC,
                  _W_SPEC, _B_SPEC, _W_SPEC, _W_SPEC, _B_SPEC],
        out_specs=_ROW_SPEC,
        out_shape=jax.ShapeDtypeStruct((N, D), jnp.float32),
    )(agg, cnt, xlo, xhi, wl, bl.reshape(1, D), wr, wo, bo.reshape(1, D))


def kernel(x_user, x_item, edge_index,
           Wl_ui0, bl_ui0, Wr_ui0, Wl_iu0, bl_iu0, Wr_iu0,
           Wl_ui1, bl_ui1, Wr_ui1, Wl_iu1, bl_iu1, Wr_iu1,
           Wu, bu, Wi, bi):
    src = edge_index[0].astype(jnp.int32)
    dst = edge_index[1].astype(jnp.int32)
    pad = jnp.full((EP - E,), N, jnp.int32)
    src_p = jnp.concatenate([src, pad]).reshape(NCT, SS, CHUNK)
    dst_p = jnp.concatenate([dst, pad]).reshape(NCT, SS, CHUNK)
    zrow = jnp.zeros((NP - N, H), jnp.float32)
    xu_lo = jnp.concatenate([x_user[:, :H], zrow])
    xu_hi = jnp.concatenate([x_user[:, H:], zrow])
    xi_lo = jnp.concatenate([x_item[:, :H], zrow])
    xi_hi = jnp.concatenate([x_item[:, H:], zrow])

    cnt = _cnt_call(src_p, dst_p)

    agg_i0 = _agg_call(src_p, dst_p, xu_lo, xu_hi)
    agg_u0 = _agg_call(dst_p, src_p, xi_lo, xi_hi)
    ih_lo, ih_hi = _dense(agg_i0, cnt, xi_lo, xi_hi, Wl_ui0, bl_ui0,
                          Wr_ui0, 16)
    uh_lo, uh_hi = _dense(agg_u0, cnt, xu_lo, xu_hi, Wl_iu0, bl_iu0,
                          Wr_iu0, 0)

    agg_i1 = _agg_call(src_p, dst_p, uh_lo, uh_hi)
    agg_u1 = _agg_call(dst_p, src_p, ih_lo, ih_hi)
    item_emb = _dense_final(agg_i1, cnt, ih_lo, ih_hi, Wl_ui1, bl_ui1,
                            Wr_ui1, Wi, bi, 16)
    user_emb = _dense_final(agg_u1, cnt, uh_lo, uh_hi, Wl_iu1, bl_iu1,
                            Wr_iu1, Wu, bu, 0)
    return (user_emb, item_emb)


# speedup vs baseline: 1.0380x; 1.0380x over previous
import functools

import jax
import jax.numpy as jnp
from jax import lax
from jax.experimental import pallas as pl
from jax.experimental.pallas import tpu as pltpu
from jax.experimental.pallas import tpu_sc as plsc

N = 100000
E = 1600000
D = 32
H = 16

NSUB = 16
GROUP = 128
GSUP = 16

NP = 100352
TR = NP // NSUB
ZR = 128
EP = 1638400
NCT = EP // (GSUP * GROUP)
NSUP = NCT // NSUB

BR = 2048
_MESH = plsc.VectorSubcoreMesh(
    core_axis_name="c", subcore_axis_name="s", num_cores=2, num_subcores=NSUB
)


def _agg_body(gidx_hbm, sidx_hbm, tlo_hbm, thi_hbm, out_hbm,
              acc_sh, gi_v, si_v, rows_v, zb_v,
              semi0, semi1, semg0, semg1, semg2, semg3,
              sems0, sems1, sems2, sems3):
    c = lax.axis_index("c")
    s = lax.axis_index("s")
    semi = (semi0, semi1)
    semg = (semg0, semg1, semg2, semg3)
    sems = (sems0, sems1, sems2, sems3)

    def _zero(j, carry):
        zb_v[j, :] = jnp.zeros((16,), jnp.float32)
        return carry

    lax.fori_loop(0, ZR, _zero, 0)
    r0 = s * TR
    for t in range(TR // ZR):
        pltpu.async_copy(zb_v, acc_sh.at[pl.ds(r0 + t * ZR, ZR)], semg0)
    for t in range(TR // ZR):
        pltpu.make_async_copy(zb_v, acc_sh.at[pl.ds(r0 + t * ZR, ZR)],
                              semg0).wait()
    plsc.subcore_barrier()

    def _load_super(p, j):
        row = s * NSUP + j
        pltpu.async_copy(gidx_hbm.at[row], gi_v.at[p], semi[p])
        pltpu.async_copy(sidx_hbm.at[row], si_v.at[p], semi[p])

    def _drain_super(p):
        pltpu.make_async_copy(gidx_hbm.at[0], gi_v.at[p], semi[p]).wait()
        pltpu.make_async_copy(sidx_hbm.at[0], si_v.at[p], semi[p]).wait()

    def _fire_g(p, m):
        b = m % 8

        @pl.when(c == 0)
        def _():
            pltpu.async_copy(tlo_hbm.at[gi_v.at[p, m]], rows_v.at[b],
                             semg[m % 4])

        @pl.when(c == 1)
        def _():
            pltpu.async_copy(thi_hbm.at[gi_v.at[p, m]], rows_v.at[b],
                             semg[m % 4])

    def _drain_g(b, slot):
        pltpu.make_async_copy(tlo_hbm.at[gi_v.at[0, 0]], rows_v.at[b],
                              semg[slot]).wait()

    def _fire_s(p, m, slot):
        pltpu.async_copy(rows_v.at[m % 8], acc_sh.at[si_v.at[p, m]],
                         sems[slot], add=True)

    def _drain_s(slot):
        pltpu.make_async_copy(rows_v.at[0], acc_sh.at[si_v.at[0, 0]],
                              sems[slot]).wait()

    def _do_super(p, j, first, load_next):
        _drain_super(p)
        for m in range(GSUP):
            if m == 8 and load_next:
                _load_super(1 - p, j + 1)
            if not (first and m < 4):
                _drain_g((m - 4) % 8, m % 4)
            if not (first and m < 8):
                _drain_s(m % 4)
            if not (first and m < 4):
                if m >= 4:
                    _fire_s(p, m - 4, m % 4)
                else:
                    _fire_s(1 - p, 12 + m, m % 4)
            _fire_g(p, m)

    _load_super(0, 0)
    _do_super(0, 0, True, True)

    def _iter(it, carry):
        _do_super(1, 2 * it + 1, False, True)
        _do_super(0, 2 * it + 2, False, True)
        return carry

    lax.fori_loop(0, (NSUP - 2) // 2, _iter, 0)
    _do_super(1, NSUP - 1, False, False)
    for m in range(4):
        _drain_g((12 + m) % 8, m % 4)
        _fire_s(1, 12 + m, m % 4)
    for m in range(8):
        _drain_s(m % 4)
    plsc.subcore_barrier()

    @pl.when(c == 0)
    def _():
        pltpu.sync_copy(acc_sh.at[pl.ds(r0, TR)],
                        out_hbm.at[pl.ds(r0, TR), pl.ds(0, 16)])

    @pl.when(c == 1)
    def _():
        pltpu.sync_copy(acc_sh.at[pl.ds(r0, TR)],
                        out_hbm.at[pl.ds(r0, TR), pl.ds(16, 16)])


_agg_call = functools.partial(
    pl.kernel,
    out_type=jax.ShapeDtypeStruct((NP, D), jnp.float32),
    mesh=_MESH,
    scratch_types=[
        pltpu.VMEM_SHARED((NP, H), jnp.float32),
        pltpu.VMEM((2, GSUP, GROUP), jnp.int32),
        pltpu.VMEM((2, GSUP, GROUP), jnp.int32),
        pltpu.VMEM((8, GROUP, H), jnp.float32),
        pltpu.VMEM((ZR, H), jnp.float32),
        pltpu.SemaphoreType.DMA,
        pltpu.SemaphoreType.DMA,
        pltpu.SemaphoreType.DMA,
        pltpu.SemaphoreType.DMA,
        pltpu.SemaphoreType.DMA,
        pltpu.SemaphoreType.DMA,
        pltpu.SemaphoreType.DMA,
        pltpu.SemaphoreType.DMA,
        pltpu.SemaphoreType.DMA,
        pltpu.SemaphoreType.DMA,
    ],
    compiler_params=pltpu.CompilerParams(use_tc_tiling_on_sc=False),
)(_agg_body)


def _cnt_body(su_hbm, sd_hbm, out_hbm, acc_sh, si_v, ones_v, zb_v,
              semi0, semi1, sems0, sems1, sems2, sems3):
    c = lax.axis_index("c")
    s = lax.axis_index("s")
    semi = (semi0, semi1)
    sems = (sems0, sems1, sems2, sems3)

    def _zero(j, carry):
        zb_v[j, :] = jnp.zeros((16,), jnp.float32)
        return carry

    lax.fori_loop(0, ZR, _zero, 0)

    def _one(j, carry):
        ones_v[j, :] = jnp.ones((16,), jnp.float32)
        return carry

    lax.fori_loop(0, 128, _one, 0)
    r0 = s * TR
    for t in range(TR // ZR):
        pltpu.async_copy(zb_v, acc_sh.at[pl.ds(r0 + t * ZR, ZR)], semi0)
    for t in range(TR // ZR):
        pltpu.make_async_copy(zb_v, acc_sh.at[pl.ds(r0 + t * ZR, ZR)],
                              semi0).wait()
    plsc.subcore_barrier()

    def _load_super(p, j):
        row = s * NSUP + j

        @pl.when(c == 0)
        def _():
            pltpu.async_copy(su_hbm.at[row], si_v.at[p], semi[p])

        @pl.when(c == 1)
        def _():
            pltpu.async_copy(sd_hbm.at[row], si_v.at[p], semi[p])

    def _drain_super(p):
        pltpu.make_async_copy(su_hbm.at[0], si_v.at[p], semi[p]).wait()

    def _fire_s(p, m):
        pltpu.async_copy(ones_v, acc_sh.at[si_v.at[p, m]], sems[m % 4],
                         add=True)

    def _drain_s(slot):
        pltpu.make_async_copy(ones_v, acc_sh.at[si_v.at[0, 0]],
                              sems[slot]).wait()

    def _do_super(p, j, first, load_next):
        _drain_super(p)
        for m in range(GSUP):
            if m == 8 and load_next:
                _load_super(1 - p, j + 1)
            if not (first and m < 4):
                _drain_s(m % 4)
            _fire_s(p, m)

    _load_super(0, 0)
    _do_super(0, 0, True, True)

    def _iter(it, carry):
        _do_super(1, 2 * it + 1, False, True)
        _do_super(0, 2 * it + 2, False, True)
        return carry

    lax.fori_loop(0, (NSUP - 2) // 2, _iter, 0)
    _do_super(1, NSUP - 1, False, False)
    for m in range(4):
        _drain_s(m % 4)
    plsc.subcore_barrier()

    @pl.when(c == 0)
    def _():
        pltpu.sync_copy(acc_sh.at[pl.ds(r0, TR)],
                        out_hbm.at[pl.ds(r0, TR), pl.ds(0, 16)])

    @pl.when(c == 1)
    def _():
        pltpu.sync_copy(acc_sh.at[pl.ds(r0, TR)],
                        out_hbm.at[pl.ds(r0, TR), pl.ds(16, 16)])


_cnt_call = functools.partial(
    pl.kernel,
    out_type=jax.ShapeDtypeStruct((NP, D), jnp.float32),
    mesh=_MESH,
    scratch_types=[
        pltpu.VMEM_SHARED((NP, H), jnp.float32),
        pltpu.VMEM((2, GSUP, GROUP), jnp.int32),
        pltpu.VMEM((GROUP, H), jnp.float32),
        pltpu.VMEM((ZR, H), jnp.float32),
        pltpu.SemaphoreType.DMA,
        pltpu.SemaphoreType.DMA,
        pltpu.SemaphoreType.DMA,
        pltpu.SemaphoreType.DMA,
        pltpu.SemaphoreType.DMA,
        pltpu.SemaphoreType.DMA,
    ],
    compiler_params=pltpu.CompilerParams(use_tc_tiling_on_sc=False),
)(_cnt_body)


def _dense_body(agg_ref, cnt_ref, xlo_ref, xhi_ref, wl_ref, bl_ref, wr_ref,
                lo_ref, hi_ref, *, col):
    cnt = jnp.maximum(cnt_ref[:, col:col + 1], 1.0)
    agg = agg_ref[...] / cnt
    xd = jnp.concatenate([xlo_ref[...], xhi_ref[...]], axis=1)
    o = (jnp.dot(agg, wl_ref[...], preferred_element_type=jnp.float32)
         + bl_ref[...]
         + jnp.dot(xd, wr_ref[...], preferred_element_type=jnp.float32))
    nrm = jnp.sqrt(jnp.sum(o * o, axis=-1, keepdims=True))
    h = jnp.maximum(o / jnp.maximum(nrm, 1e-12), 0.0)
    lo_ref[...] = h[:, :H]
    hi_ref[...] = h[:, H:]


def _dense_final_body(agg_ref, cnt_ref, xlo_ref, xhi_ref, wl_ref, bl_ref,
                      wr_ref, wo_ref, bo_ref, out_ref, *, col):
    cnt = jnp.maximum(cnt_ref[:, col:col + 1], 1.0)
    agg = agg_ref[...] / cnt
    xd = jnp.concatenate([xlo_ref[...], xhi_ref[...]], axis=1)
    o = (jnp.dot(agg, wl_ref[...], preferred_element_type=jnp.float32)
         + bl_ref[...]
         + jnp.dot(xd, wr_ref[...], preferred_element_type=jnp.float32))
    nrm = jnp.sqrt(jnp.sum(o * o, axis=-1, keepdims=True))
    h = jnp.maximum(o / jnp.maximum(nrm, 1e-12), 0.0)
    e = jnp.dot(h, wo_ref[...], preferred_element_type=jnp.float32) + bo_ref[...]
    nrm2 = jnp.sqrt(jnp.sum(e * e, axis=-1, keepdims=True))
    out_ref[...] = e / jnp.maximum(nrm2, 1e-12)


_ROW_SPEC = pl.BlockSpec((BR, D), lambda i: (i, 0))
_HALF_SPEC = pl.BlockSpec((BR, H), lambda i: (i, 0))
_W_SPEC = pl.BlockSpec((D, D), lambda i: (0, 0))
_B_SPEC = pl.BlockSpec((1, D), lambda i: (0, 0))


def _dense(agg, cnt, xlo, xhi, wl, bl, wr, col):
    return pl.pallas_call(
        functools.partial(_dense_body, col=col),
        grid=(NP // BR,),
        in_specs=[_ROW_SPEC, _ROW_SPEC, _HALF_SPEC, _HALF_SPEC,
                  _W_SPEC, _B_SPEC, _W_SPEC],
        out_specs=[_HALF_SPEC, _HALF_SPEC],
        out_shape=[jax.ShapeDtypeStruct((NP, H), jnp.float32),
                   jax.ShapeDtypeStruct((NP, H), jnp.float32)],
    )(agg, cnt, xlo, xhi, wl, bl.reshape(1, D), wr)


def _dense_final(agg, cnt, xlo, xhi, wl, bl, wr, wo, bo, col):
    return pl.pallas_call(
        functools.partial(_dense_final_body, col=col),
        grid=(NP // BR,),
        in_specs=[_ROW_SPEC, _ROW_SPEC, _HALF_SPEC, _HALF_SPEC,
                  _W_SPEC, _B_SPEC, _W_SPEC, _W_SPEC, _B_SPEC],
        out_specs=_ROW_SPEC,
        out_shape=jax.ShapeDtypeStruct((N, D), jnp.float32),
    )(agg, cnt, xlo, xhi, wl, bl.reshape(1, D), wr, wo, bo.reshape(1, D))


def kernel(x_user, x_item, edge_index,
           Wl_ui0, bl_ui0, Wr_ui0, Wl_iu0, bl_iu0, Wr_iu0,
           Wl_ui1, bl_ui1, Wr_ui1, Wl_iu1, bl_iu1, Wr_iu1,
           Wu, bu, Wi, bi):
    src = edge_index[0].astype(jnp.int32)
    dst = edge_index[1].astype(jnp.int32)
    pad = jnp.full((EP - E,), N, jnp.int32)
    src_p = jnp.concatenate([src, pad]).reshape(NCT, GSUP, GROUP)
    dst_p = jnp.concatenate([dst, pad]).reshape(NCT, GSUP, GROUP)
    zrow = jnp.zeros((NP - N, H), jnp.float32)
    xu_lo = jnp.concatenate([x_user[:, :H], zrow])
    xu_hi = jnp.concatenate([x_user[:, H:], zrow])
    xi_lo = jnp.concatenate([x_item[:, :H], zrow])
    xi_hi = jnp.concatenate([x_item[:, H:], zrow])

    cnt = _cnt_call(src_p, dst_p)

    agg_i0 = _agg_call(src_p, dst_p, xu_lo, xu_hi)
    agg_u0 = _agg_call(dst_p, src_p, xi_lo, xi_hi)
    ih_lo, ih_hi = _dense(agg_i0, cnt, xi_lo, xi_hi, Wl_ui0, bl_ui0,
                          Wr_ui0, 16)
    uh_lo, uh_hi = _dense(agg_u0, cnt, xu_lo, xu_hi, Wl_iu0, bl_iu0,
                          Wr_iu0, 0)

    agg_i1 = _agg_call(src_p, dst_p, uh_lo, uh_hi)
    agg_u1 = _agg_call(dst_p, src_p, ih_lo, ih_hi)
    item_emb = _dense_final(agg_i1, cnt, ih_lo, ih_hi, Wl_ui1, bl_ui1,
                            Wr_ui1, Wi, bi, 16)
    user_emb = _dense_final(agg_u1, cnt, uh_lo, uh_hi, Wl_iu1, bl_iu1,
                            Wr_iu1, Wu, bu, 0)
    return (user_emb, item_emb)


# speedup vs baseline: 1.1293x; 1.0880x over previous
import functools

import jax
import jax.numpy as jnp
from jax import lax
from jax.experimental import pallas as pl
from jax.experimental.pallas import tpu as pltpu
from jax.experimental.pallas import tpu_sc as plsc

N = 100000
E = 1600000
D = 32
H = 16

NSUB = 16
GROUP = 128
GSUP = 16

NP = 100352
TR = NP // NSUB
ZR = 128
EP = 1638400
NCT = EP // (GSUP * GROUP)
NSUP = NCT // NSUB

BR = 2048
_MESH = plsc.VectorSubcoreMesh(
    core_axis_name="c", subcore_axis_name="s", num_cores=2, num_subcores=NSUB
)


def _agg_body(gidx_hbm, sidx_hbm, tbl_hbm, out_hbm,
              acc_sh, gi_v, si_v, rows_v, zb_v,
              semi0, semi1, semg0, semg1, semg2, semg3,
              sems0, sems1, sems2, sems3):
    c = lax.axis_index("c")
    s = lax.axis_index("s")
    semi = (semi0, semi1)
    semg = (semg0, semg1, semg2, semg3)
    sems = (sems0, sems1, sems2, sems3)

    def _zero(j, carry):
        zb_v[j, :] = jnp.zeros((16,), jnp.float32)
        return carry

    lax.fori_loop(0, ZR, _zero, 0)
    r0 = s * TR
    for t in range(TR // ZR):
        pltpu.async_copy(zb_v, acc_sh.at[pl.ds(r0 + t * ZR, ZR)], semg0)
    for t in range(TR // ZR):
        pltpu.make_async_copy(zb_v, acc_sh.at[pl.ds(r0 + t * ZR, ZR)],
                              semg0).wait()
    plsc.subcore_barrier()

    def _load_super(p, j):
        row = s * NSUP + j
        pltpu.async_copy(gidx_hbm.at[row], gi_v.at[p], semi[p])
        pltpu.async_copy(sidx_hbm.at[row], si_v.at[p], semi[p])

    def _drain_super(p):
        pltpu.make_async_copy(gidx_hbm.at[0], gi_v.at[p], semi[p]).wait()
        pltpu.make_async_copy(sidx_hbm.at[0], si_v.at[p], semi[p]).wait()

    def _fire_g(p, m):
        pltpu.async_copy(tbl_hbm.at[c].at[gi_v.at[p, m]], rows_v.at[m % 8],
                         semg[m % 4])

    def _drain_g(b, slot):
        pltpu.make_async_copy(tbl_hbm.at[c].at[gi_v.at[0, 0]], rows_v.at[b],
                              semg[slot]).wait()

    def _fire_s(p, m, slot):
        pltpu.async_copy(rows_v.at[m % 8], acc_sh.at[si_v.at[p, m]],
                         sems[slot], add=True)

    def _drain_s(slot):
        pltpu.make_async_copy(rows_v.at[0], acc_sh.at[si_v.at[0, 0]],
                              sems[slot]).wait()

    def _do_super(p, j, first, load_next):
        _drain_super(p)
        for m in range(GSUP):
            if m == 8 and load_next:
                _load_super(1 - p, j + 1)
            if not (first and m < 4):
                _drain_g((m - 4) % 8, m % 4)
            if not (first and m < 8):
                _drain_s(m % 4)
            if not (first and m < 4):
                if m >= 4:
                    _fire_s(p, m - 4, m % 4)
                else:
                    _fire_s(1 - p, 12 + m, m % 4)
            _fire_g(p, m)

    _load_super(0, 0)
    _do_super(0, 0, True, True)

    def _iter(it, carry):
        _do_super(1, 2 * it + 1, False, True)
        _do_super(0, 2 * it + 2, False, True)
        return carry

    lax.fori_loop(0, (NSUP - 2) // 2, _iter, 0)
    _do_super(1, NSUP - 1, False, False)
    for m in range(4):
        _drain_g((12 + m) % 8, m % 4)
        _fire_s(1, 12 + m, m % 4)
    for m in range(8):
        _drain_s(m % 4)
    plsc.subcore_barrier()

    @pl.when(c == 0)
    def _():
        pltpu.sync_copy(acc_sh.at[pl.ds(r0, TR)],
                        out_hbm.at[pl.ds(r0, TR), pl.ds(0, 16)])

    @pl.when(c == 1)
    def _():
        pltpu.sync_copy(acc_sh.at[pl.ds(r0, TR)],
                        out_hbm.at[pl.ds(r0, TR), pl.ds(16, 16)])


_agg_call = functools.partial(
    pl.kernel,
    out_type=jax.ShapeDtypeStruct((NP, D), jnp.float32),
    mesh=_MESH,
    scratch_types=[
        pltpu.VMEM_SHARED((NP, H), jnp.float32),
        pltpu.VMEM((2, GSUP, GROUP), jnp.int32),
        pltpu.VMEM((2, GSUP, GROUP), jnp.int32),
        pltpu.VMEM((8, GROUP, H), jnp.float32),
        pltpu.VMEM((ZR, H), jnp.float32),
        pltpu.SemaphoreType.DMA,
        pltpu.SemaphoreType.DMA,
        pltpu.SemaphoreType.DMA,
        pltpu.SemaphoreType.DMA,
        pltpu.SemaphoreType.DMA,
        pltpu.SemaphoreType.DMA,
        pltpu.SemaphoreType.DMA,
        pltpu.SemaphoreType.DMA,
        pltpu.SemaphoreType.DMA,
        pltpu.SemaphoreType.DMA,
    ],
    compiler_params=pltpu.CompilerParams(use_tc_tiling_on_sc=False),
)(_agg_body)


def _cnt_body(su_hbm, sd_hbm, out_hbm, acc_sh, si_v, ones_v, zb_v,
              semi0, semi1, sems0, sems1, sems2, sems3):
    c = lax.axis_index("c")
    s = lax.axis_index("s")
    semi = (semi0, semi1)
    sems = (sems0, sems1, sems2, sems3)

    def _zero(j, carry):
        zb_v[j, :] = jnp.zeros((16,), jnp.float32)
        return carry

    lax.fori_loop(0, ZR, _zero, 0)

    def _one(j, carry):
        ones_v[j, :] = jnp.ones((16,), jnp.float32)
        return carry

    lax.fori_loop(0, 128, _one, 0)
    r0 = s * TR
    for t in range(TR // ZR):
        pltpu.async_copy(zb_v, acc_sh.at[pl.ds(r0 + t * ZR, ZR)], semi0)
    for t in range(TR // ZR):
        pltpu.make_async_copy(zb_v, acc_sh.at[pl.ds(r0 + t * ZR, ZR)],
                              semi0).wait()
    plsc.subcore_barrier()

    def _load_super(p, j):
        row = s * NSUP + j

        @pl.when(c == 0)
        def _():
            pltpu.async_copy(su_hbm.at[row], si_v.at[p], semi[p])

        @pl.when(c == 1)
        def _():
            pltpu.async_copy(sd_hbm.at[row], si_v.at[p], semi[p])

    def _drain_super(p):
        pltpu.make_async_copy(su_hbm.at[0], si_v.at[p], semi[p]).wait()

    def _fire_s(p, m):
        pltpu.async_copy(ones_v, acc_sh.at[si_v.at[p, m]], sems[m % 4],
                         add=True)

    def _drain_s(slot):
        pltpu.make_async_copy(ones_v, acc_sh.at[si_v.at[0, 0]],
                              sems[slot]).wait()

    def _do_super(p, j, first, load_next):
        _drain_super(p)
        for m in range(GSUP):
            if m == 8 and load_next:
                _load_super(1 - p, j + 1)
            if not (first and m < 4):
                _drain_s(m % 4)
            _fire_s(p, m)

    _load_super(0, 0)
    _do_super(0, 0, True, True)

    def _iter(it, carry):
        _do_super(1, 2 * it + 1, False, True)
        _do_super(0, 2 * it + 2, False, True)
        return carry

    lax.fori_loop(0, (NSUP - 2) // 2, _iter, 0)
    _do_super(1, NSUP - 1, False, False)
    for m in range(4):
        _drain_s(m % 4)
    plsc.subcore_barrier()

    @pl.when(c == 0)
    def _():
        pltpu.sync_copy(acc_sh.at[pl.ds(r0, TR)],
                        out_hbm.at[pl.ds(r0, TR), pl.ds(0, 16)])

    @pl.when(c == 1)
    def _():
        pltpu.sync_copy(acc_sh.at[pl.ds(r0, TR)],
                        out_hbm.at[pl.ds(r0, TR), pl.ds(16, 16)])


_cnt_call = functools.partial(
    pl.kernel,
    out_type=jax.ShapeDtypeStruct((NP, D), jnp.float32),
    mesh=_MESH,
    scratch_types=[
        pltpu.VMEM_SHARED((NP, H), jnp.float32),
        pltpu.VMEM((2, GSUP, GROUP), jnp.int32),
        pltpu.VMEM((GROUP, H), jnp.float32),
        pltpu.VMEM((ZR, H), jnp.float32),
        pltpu.SemaphoreType.DMA,
        pltpu.SemaphoreType.DMA,
        pltpu.SemaphoreType.DMA,
        pltpu.SemaphoreType.DMA,
        pltpu.SemaphoreType.DMA,
        pltpu.SemaphoreType.DMA,
    ],
    compiler_params=pltpu.CompilerParams(use_tc_tiling_on_sc=False),
)(_cnt_body)


def _dense_body(agg_ref, cnt_ref, xd_ref, wl_ref, bl_ref, wr_ref,
                out_ref, *, col):
    cnt = jnp.maximum(cnt_ref[:, col:col + 1], 1.0)
    agg = agg_ref[...] / cnt
    xd = jnp.concatenate([xd_ref[0], xd_ref[1]], axis=1)
    o = (jnp.dot(agg, wl_ref[...], preferred_element_type=jnp.float32)
         + bl_ref[...]
         + jnp.dot(xd, wr_ref[...], preferred_element_type=jnp.float32))
    nrm = jnp.sqrt(jnp.sum(o * o, axis=-1, keepdims=True))
    h = jnp.maximum(o / jnp.maximum(nrm, 1e-12), 0.0)
    out_ref[0] = h[:, :H]
    out_ref[1] = h[:, H:]


def _dense_final_body(agg_ref, cnt_ref, xd_ref, wl_ref, bl_ref,
                      wr_ref, wo_ref, bo_ref, out_ref, *, col):
    cnt = jnp.maximum(cnt_ref[:, col:col + 1], 1.0)
    agg = agg_ref[...] / cnt
    xd = jnp.concatenate([xd_ref[0], xd_ref[1]], axis=1)
    o = (jnp.dot(agg, wl_ref[...], preferred_element_type=jnp.float32)
         + bl_ref[...]
         + jnp.dot(xd, wr_ref[...], preferred_element_type=jnp.float32))
    nrm = jnp.sqrt(jnp.sum(o * o, axis=-1, keepdims=True))
    h = jnp.maximum(o / jnp.maximum(nrm, 1e-12), 0.0)
    e = jnp.dot(h, wo_ref[...], preferred_element_type=jnp.float32) + bo_ref[...]
    nrm2 = jnp.sqrt(jnp.sum(e * e, axis=-1, keepdims=True))
    out_ref[...] = e / jnp.maximum(nrm2, 1e-12)


_ROW_SPEC = pl.BlockSpec((BR, D), lambda i: (i, 0))
_TBL_SPEC = pl.BlockSpec((2, BR, H), lambda i: (0, i, 0))
_W_SPEC = pl.BlockSpec((D, D), lambda i: (0, 0))
_B_SPEC = pl.BlockSpec((1, D), lambda i: (0, 0))


def _dense(agg, cnt, xd, wl, bl, wr, col):
    return pl.pallas_call(
        functools.partial(_dense_body, col=col),
        grid=(NP // BR,),
        in_specs=[_ROW_SPEC, _ROW_SPEC, _TBL_SPEC, _W_SPEC, _B_SPEC, _W_SPEC],
        out_specs=_TBL_SPEC,
        out_shape=jax.ShapeDtypeStruct((2, NP, H), jnp.float32),
    )(agg, cnt, xd, wl, bl.reshape(1, D), wr)


def _dense_final(agg, cnt, xd, wl, bl, wr, wo, bo, col):
    return pl.pallas_call(
        functools.partial(_dense_final_body, col=col),
        grid=(NP // BR,),
        in_specs=[_ROW_SPEC, _ROW_SPEC, _TBL_SPEC,
                  _W_SPEC, _B_SPEC, _W_SPEC, _W_SPEC, _B_SPEC],
        out_specs=_ROW_SPEC,
        out_shape=jax.ShapeDtypeStruct((N, D), jnp.float32),
    )(agg, cnt, xd, wl, bl.reshape(1, D), wr, wo, bo.reshape(1, D))


def kernel(x_user, x_item, edge_index,
           Wl_ui0, bl_ui0, Wr_ui0, Wl_iu0, bl_iu0, Wr_iu0,
           Wl_ui1, bl_ui1, Wr_ui1, Wl_iu1, bl_iu1, Wr_iu1,
           Wu, bu, Wi, bi):
    src = edge_index[0].astype(jnp.int32)
    dst = edge_index[1].astype(jnp.int32)
    pad = jnp.full((EP - E,), N, jnp.int32)
    src_p = jnp.concatenate([src, pad]).reshape(NCT, GSUP, GROUP)
    dst_p = jnp.concatenate([dst, pad]).reshape(NCT, GSUP, GROUP)
    zrow = jnp.zeros((2, NP - N, H), jnp.float32)
    xu_t = jnp.concatenate(
        [jnp.stack([x_user[:, :H], x_user[:, H:]]), zrow], axis=1)
    xi_t = jnp.concatenate(
        [jnp.stack([x_item[:, :H], x_item[:, H:]]), zrow], axis=1)

    cnt = _cnt_call(src_p, dst_p)

    agg_i0 = _agg_call(src_p, dst_p, xu_t)
    agg_u0 = _agg_call(dst_p, src_p, xi_t)
    ih_t = _dense(agg_i0, cnt, xi_t, Wl_ui0, bl_ui0, Wr_ui0, 16)
    uh_t = _dense(agg_u0, cnt, xu_t, Wl_iu0, bl_iu0, Wr_iu0, 0)

    agg_i1 = _agg_call(src_p, dst_p, uh_t)
    agg_u1 = _agg_call(dst_p, src_p, ih_t)
    item_emb = _dense_final(agg_i1, cnt, ih_t, Wl_ui1, bl_ui1,
                            Wr_ui1, Wi, bi, 16)
    user_emb = _dense_final(agg_u1, cnt, uh_t, Wl_iu1, bl_iu1,
                            Wr_iu1, Wu, bu, 0)
    return (user_emb, item_emb)
